# baseline (speedup 1.0000x reference)
import functools

import jax
import jax.numpy as jnp
from jax import lax
from jax.experimental import pallas as pl
from jax.experimental.pallas import tpu as pltpu
from jax.experimental.pallas import tpu_sc as plsc

B = 16
T_IN = 512
D = 256
T_OUT = 4096

L = 16
NC = 2
NS = 16
HALF = T_OUT // 2
C = 32
W = C + 8
SZ = (16, 24, 32, W)
NCH = HALF // C


def _body(inp2d, tl, out, x_v, e_v, off_v, w_v, rel_v,
          y0b, y1b, y2b, y3b, o0, o1, o2, o3, start_m, cls_m,
          sg0, sg1, sg2, sg3, so0, so1, so2, so3):
    wid = lax.axis_index("s") * NC + lax.axis_index("c")
    b = wid // 2
    half = wid % 2
    t0 = half * HALF

    pltpu.sync_copy(tl, x_v)

    zeros16 = jnp.zeros((L,), jnp.int32)
    ones16 = jnp.ones((L,), jnp.int32)
    iota16 = lax.iota(jnp.int32, L)

    nch1 = HALF // L

    @plsc.parallel_loop(0, nch1, 1, unroll=4)
    def zero_body(j):
        e_v[pl.ds(j * L, L)] = zeros16

    @plsc.parallel_loop(0, T_IN // L, 1, unroll=2, carry=zeros16)
    def acc_v(j, accv):
        xv = x_v[pl.ds(j * L, L)]
        m = (xv >= t0) & (xv < t0 + HALF)
        idx = jnp.clip(xv - t0, 0, HALF - 1)
        plsc.store_scatter(e_v, [idx], ones16, mask=m)
        return accv + (xv < t0).astype(jnp.int32)

    acc0 = jnp.sum(acc_v)

    boff = b * T_IN

    @plsc.parallel_loop(0, nch1, 1, unroll=2, carry=acc0)
    def ind_body(j, acc):
        ev = e_v[pl.ds(j * L, L)]
        c = plsc.cumsum(ev) + acc
        ind = jnp.minimum(c - 1, T_IN - 2)
        x0 = plsc.load_gather(x_v, [ind])
        x1 = plsc.load_gather(x_v, [ind + 1])
        tv = (iota16 + (t0 + j * L)).astype(jnp.float32)
        w_v[pl.ds(j * L, L)] = (tv - x0.astype(jnp.float32)) / (
            (x1 - x0).astype(jnp.float32))
        off_v[pl.ds(j * L, L)] = ind
        return acc + jnp.sum(ev)

    ybufs = (y0b, y1b, y2b, y3b)
    gsems = (sg0, sg1, sg2, sg3)
    obufs, osems = (o0, o1, o2, o3), (so0, so1, so2, so3)

    @plsc.parallel_loop(0, NCH, 1, unroll=2)
    def meta_body(i):
        o = i * C
        offs = [off_v[pl.ds(o + u * L, L)] for u in range(C // L)]
        start = jnp.minimum((jnp.min(offs[0]) // 8) * 8, T_IN - W)
        for u in range(C // L):
            rel_v[pl.ds(o + u * L, L)] = offs[u] - start
        span = jnp.max(offs[-1]) + 2 - start
        start_m[i] = start
        cls_m[i] = ((span > SZ[0]).astype(jnp.int32)
                    + (span > SZ[1]).astype(jnp.int32)
                    + (span > SZ[2]).astype(jnp.int32))

    def issue_gather(it, yr, sem):
        start = pl.multiple_of(start_m[it], 8)
        scid = cls_m[it]
        for ci, sz in enumerate(SZ):
            @pl.when(scid == ci)
            def _():
                pltpu.async_copy(
                    inp2d.at[pl.ds(boff + start, sz)],
                    yr.at[pl.ds(0, sz)], sem)

    def wait_gather(it, yr, sem):
        scid = cls_m[it]
        for ci, sz in enumerate(SZ):
            @pl.when(scid == ci)
            def _():
                pltpu.make_async_copy(
                    inp2d.at[pl.ds(boff, sz)], yr.at[pl.ds(0, sz)],
                    sem).wait()

    def compute(it, yr, orf):
        o = it * C

        @plsc.parallel_loop(0, C, 1, unroll=2)
        def row_body(r):
            wv = plsc.load_gather(w_v, [zeros16 + (o + r)])
            rel = rel_v[pl.ds(o + r, L)][0]
            for cg in range(D // L):
                y0 = yr[rel, pl.ds(cg * L, L)]
                y1 = yr[rel + 1, pl.ds(cg * L, L)]
                orf[r, pl.ds(cg * L, L)] = y0 + wv * (y1 - y0)

    for p in range(3):
        issue_gather(p, ybufs[p], gsems[p])

    def outer(it4, c):
        for u in range(4):
            it = it4 * 4 + u
            ns = (u + 3) % 4
            os_ = u

            @pl.when(it + 3 < NCH)
            def _():
                issue_gather(it + 3, ybufs[ns], gsems[ns])

            wait_gather(it, ybufs[u], gsems[u])

            @pl.when(it >= 4)
            def _():
                pltpu.make_async_copy(
                    obufs[os_], out.at[b, pl.ds(t0, C)], osems[os_]).wait()

            compute(it, ybufs[u], obufs[os_])
            pltpu.async_copy(
                obufs[os_], out.at[b, pl.ds(t0 + it * C, C)], osems[os_])
        return c

    lax.fori_loop(0, NCH // 4, outer, 0)

    pltpu.make_async_copy(o0, out.at[b, pl.ds(t0, C)], so0).wait()
    pltpu.make_async_copy(o1, out.at[b, pl.ds(t0, C)], so1).wait()
    pltpu.make_async_copy(o2, out.at[b, pl.ds(t0, C)], so2).wait()
    pltpu.make_async_copy(o3, out.at[b, pl.ds(t0, C)], so3).wait()


_interp = functools.partial(
    pl.kernel,
    out_type=jax.ShapeDtypeStruct((B, T_OUT, D), jnp.float32),
    mesh=plsc.VectorSubcoreMesh(core_axis_name="c", subcore_axis_name="s"),
    compiler_params=pltpu.CompilerParams(needs_layout_passes=False),
    scratch_types=[
        pltpu.VMEM((T_IN,), jnp.int32),
        pltpu.VMEM((HALF,), jnp.int32),
        pltpu.VMEM((HALF,), jnp.int32),
        pltpu.VMEM((HALF,), jnp.float32),
        pltpu.VMEM((HALF + L,), jnp.int32),
        pltpu.VMEM((W, D), jnp.float32),
        pltpu.VMEM((W, D), jnp.float32),
        pltpu.VMEM((W, D), jnp.float32),
        pltpu.VMEM((W, D), jnp.float32),
        pltpu.VMEM((C, D), jnp.float32),
        pltpu.VMEM((C, D), jnp.float32),
        pltpu.VMEM((C, D), jnp.float32),
        pltpu.VMEM((C, D), jnp.float32),
        pltpu.SMEM((NCH,), jnp.int32),
        pltpu.SMEM((NCH,), jnp.int32),
        pltpu.SemaphoreType.DMA,
        pltpu.SemaphoreType.DMA,
        pltpu.SemaphoreType.DMA,
        pltpu.SemaphoreType.DMA,
        pltpu.SemaphoreType.DMA,
        pltpu.SemaphoreType.DMA,
        pltpu.SemaphoreType.DMA,
        pltpu.SemaphoreType.DMA,
    ],
)(_body)


def kernel(inp, inp_timeline):
    return _interp(inp.reshape(B * T_IN, D), inp_timeline)


# speedup vs baseline: 1.2905x; 1.2905x over previous
import functools

import jax
import jax.numpy as jnp
from jax import lax
from jax.experimental import pallas as pl
from jax.experimental.pallas import tpu as pltpu
from jax.experimental.pallas import tpu_sc as plsc

B = 16
T_IN = 512
D = 256
T_OUT = 4096

L = 16
NC = 2
NS = 16
HALF = T_OUT // 2
C = 64
W = C + 8
SZ = (16, 32, 48, W)
NCH = HALF // C


def _body(inp2d, tl, out, x_v, e_v, off_v, w_v, rel_v,
          y0b, y1b, y2b, y3b, o0, o1, start_m, cls_m,
          sg0, sg1, sg2, sg3, so0, so1):
    wid = lax.axis_index("s") * NC + lax.axis_index("c")
    b = wid // 2
    half = wid % 2
    t0 = half * HALF

    pltpu.sync_copy(tl, x_v)

    zeros16 = jnp.zeros((L,), jnp.int32)
    ones16 = jnp.ones((L,), jnp.int32)
    iota16 = lax.iota(jnp.int32, L)

    nch1 = HALF // L

    @plsc.parallel_loop(0, nch1, 1, unroll=4)
    def zero_body(j):
        e_v[pl.ds(j * L, L)] = zeros16

    @plsc.parallel_loop(0, T_IN // L, 1, unroll=2, carry=zeros16)
    def acc_v(j, accv):
        xv = x_v[pl.ds(j * L, L)]
        m = (xv >= t0) & (xv < t0 + HALF)
        idx = jnp.clip(xv - t0, 0, HALF - 1)
        plsc.store_scatter(e_v, [idx], ones16, mask=m)
        return accv + (xv < t0).astype(jnp.int32)

    acc0 = jnp.sum(acc_v)

    boff = b * T_IN

    @plsc.parallel_loop(0, nch1, 1, unroll=2, carry=acc0)
    def ind_body(j, acc):
        ev = e_v[pl.ds(j * L, L)]
        c = plsc.cumsum(ev) + acc
        ind = jnp.minimum(c - 1, T_IN - 2)
        x0 = plsc.load_gather(x_v, [ind])
        x1 = plsc.load_gather(x_v, [ind + 1])
        tv = (iota16 + (t0 + j * L)).astype(jnp.float32)
        w_v[pl.ds(j * L, L)] = (tv - x0.astype(jnp.float32)) / (
            (x1 - x0).astype(jnp.float32))
        off_v[pl.ds(j * L, L)] = ind
        return acc + jnp.sum(ev)

    ybufs = (y0b, y1b, y2b, y3b)
    gsems = (sg0, sg1, sg2, sg3)
    obufs, osems = (o0, o1), (so0, so1)

    @plsc.parallel_loop(0, NCH, 1, unroll=2)
    def meta_body(i):
        o = i * C
        offs = [off_v[pl.ds(o + u * L, L)] for u in range(C // L)]
        start = jnp.minimum((jnp.min(offs[0]) // 8) * 8, T_IN - W)
        for u in range(C // L):
            rel_v[pl.ds(o + u * L, L)] = offs[u] - start
        span = jnp.max(offs[-1]) + 2 - start
        start_m[i] = start
        cls_m[i] = ((span > SZ[0]).astype(jnp.int32)
                    + (span > SZ[1]).astype(jnp.int32)
                    + (span > SZ[2]).astype(jnp.int32))

    def issue_gather(it, yr, sem):
        start = pl.multiple_of(start_m[it], 8)
        scid = cls_m[it]
        for ci, sz in enumerate(SZ):
            @pl.when(scid == ci)
            def _():
                pltpu.async_copy(
                    inp2d.at[pl.ds(boff + start, sz)],
                    yr.at[pl.ds(0, sz)], sem)

    def wait_gather(it, yr, sem):
        scid = cls_m[it]
        for ci, sz in enumerate(SZ):
            @pl.when(scid == ci)
            def _():
                pltpu.make_async_copy(
                    inp2d.at[pl.ds(boff, sz)], yr.at[pl.ds(0, sz)],
                    sem).wait()

    def compute(it, yr, orf):
        o = it * C

        @plsc.parallel_loop(0, C, 1, unroll=2)
        def row_body(r):
            wv = plsc.load_gather(w_v, [zeros16 + (o + r)])
            rel = rel_v[pl.ds(o + r, L)][0]
            for cg in range(D // L):
                y0 = yr[rel, pl.ds(cg * L, L)]
                y1 = yr[rel + 1, pl.ds(cg * L, L)]
                orf[r, pl.ds(cg * L, L)] = y0 + wv * (y1 - y0)

    for p in range(3):
        issue_gather(p, ybufs[p], gsems[p])

    def outer(it4, c):
        for u in range(4):
            it = it4 * 4 + u
            ns = (u + 3) % 4
            os_ = u % 2

            @pl.when(it + 3 < NCH)
            def _():
                issue_gather(it + 3, ybufs[ns], gsems[ns])

            wait_gather(it, ybufs[u], gsems[u])

            @pl.when(it >= 2)
            def _():
                pltpu.make_async_copy(
                    obufs[os_], out.at[b, pl.ds(t0, C)], osems[os_]).wait()

            compute(it, ybufs[u], obufs[os_])
            pltpu.async_copy(
                obufs[os_], out.at[b, pl.ds(t0 + it * C, C)], osems[os_])
        return c

    lax.fori_loop(0, NCH // 4, outer, 0)

    pltpu.make_async_copy(o0, out.at[b, pl.ds(t0, C)], so0).wait()
    pltpu.make_async_copy(o1, out.at[b, pl.ds(t0, C)], so1).wait()


_interp = functools.partial(
    pl.kernel,
    out_type=jax.ShapeDtypeStruct((B, T_OUT, D), jnp.float32),
    mesh=plsc.VectorSubcoreMesh(core_axis_name="c", subcore_axis_name="s"),
    compiler_params=pltpu.CompilerParams(needs_layout_passes=False),
    scratch_types=[
        pltpu.VMEM((T_IN,), jnp.int32),
        pltpu.VMEM((HALF,), jnp.int32),
        pltpu.VMEM((HALF,), jnp.int32),
        pltpu.VMEM((HALF,), jnp.float32),
        pltpu.VMEM((HALF + L,), jnp.int32),
        pltpu.VMEM((W, D), jnp.float32),
        pltpu.VMEM((W, D), jnp.float32),
        pltpu.VMEM((W, D), jnp.float32),
        pltpu.VMEM((W, D), jnp.float32),
        pltpu.VMEM((C, D), jnp.float32),
        pltpu.VMEM((C, D), jnp.float32),
        pltpu.SMEM((NCH,), jnp.int32),
        pltpu.SMEM((NCH,), jnp.int32),
        pltpu.SemaphoreType.DMA,
        pltpu.SemaphoreType.DMA,
        pltpu.SemaphoreType.DMA,
        pltpu.SemaphoreType.DMA,
        pltpu.SemaphoreType.DMA,
        pltpu.SemaphoreType.DMA,
    ],
)(_body)


def kernel(inp, inp_timeline):
    return _interp(inp.reshape(B * T_IN, D), inp_timeline)
